# baseline (speedup 1.0000x reference)
import jax
import jax.numpy as jnp
from jax.experimental import pallas as pl
from jax.experimental.pallas import tpu as pltpu


def _pick_block(n, target):
    for b in range(min(n, target), 0, -1):
        if n % b == 0:
            return b
    return n


def _bdot(x, w):
    return jnp.dot(x.astype(jnp.bfloat16), w.astype(jnp.bfloat16),
                   preferred_element_type=jnp.float32)


def _mm_plain_kernel(x_ref, w_ref, o_ref):
    o_ref[...] = _bdot(x_ref[...], w_ref[...])


def _mm_tanh_post_kernel(x_ref, w_ref, p_ref, o_ref):
    acc = _bdot(x_ref[...], w_ref[...])
    o_ref[...] = jnp.dot(jnp.tanh(acc), p_ref[...],
                         preferred_element_type=jnp.float32)


def _mm_tanh_post_bias_kernel(x_ref, w_ref, p_ref, b_ref, o_ref):
    acc = _bdot(x_ref[...], w_ref[...])
    o_ref[...] = (jnp.dot(jnp.tanh(acc), p_ref[...],
                          preferred_element_type=jnp.float32)
                  + b_ref[...])


def _stage(x, w, post=None, bias=None, *, bm_target=200, interpret=False):
    m, kdim = x.shape
    h = w.shape[1]
    bm = _pick_block(m, bm_target)
    n_out = h if post is None else post.shape[1]

    in_specs = [
        pl.BlockSpec((bm, kdim), lambda i: (i, 0)),
        pl.BlockSpec((kdim, h), lambda i: (0, 0)),
    ]
    inputs = [x, w]
    if post is None:
        body = _mm_plain_kernel
    else:
        in_specs.append(pl.BlockSpec(post.shape, lambda i: (0, 0)))
        inputs.append(post)
        if bias is None:
            body = _mm_tanh_post_kernel
        else:
            in_specs.append(pl.BlockSpec(bias.shape, lambda i: (0, 0)))
            inputs.append(bias)
            body = _mm_tanh_post_bias_kernel

    return pl.pallas_call(
        body,
        grid=(m // bm,),
        in_specs=in_specs,
        out_specs=pl.BlockSpec((bm, n_out), lambda i: (i, 0)),
        out_shape=jax.ShapeDtypeStruct((m, n_out), jnp.float32),
        compiler_params=pltpu.CompilerParams(
            dimension_semantics=("arbitrary",)),
        interpret=interpret,
    )(*inputs)


def kernel(features, A0, W1, W2, Wc, bc):
    s1 = _stage(features, W1)
    s2 = _stage(A0, s1, post=W2)
    logits = _stage(A0, s2, post=Wc,
                    bias=bc.reshape(1, -1))
    return logits


# speedup vs baseline: 1.0355x; 1.0355x over previous
import functools

import jax
import jax.numpy as jnp
from jax import lax
from jax.experimental import pallas as pl
from jax.experimental.pallas import tpu as pltpu


def _pick_block(n, target):
    for b in range(min(n, target), 0, -1):
        if n % b == 0:
            return b
    return n


def _bdot(x, w):
    return jnp.dot(x.astype(jnp.bfloat16), w.astype(jnp.bfloat16),
                   preferred_element_type=jnp.float32)


def _s1_kernel(x_ref, w1_ref, o_ref):
    o_ref[...] = _bdot(x_ref[...], w1_ref[...]).astype(jnp.bfloat16)


def _fused_kernel(a_ref, s1_ref, w2_ref, wc_ref, bc_ref, o_ref, s2_ref, *,
                  nb, bm):
    i = pl.program_id(0)
    j = lax.rem(i, nb)

    @pl.when(i < nb)
    def _():
        acc = _bdot(a_ref[...], s1_ref[...])
        s2_ref[pl.ds(j * bm, bm), :] = _bdot(
            jnp.tanh(acc), w2_ref[...]).astype(jnp.bfloat16)

    @pl.when(i >= nb)
    def _():
        acc = _bdot(a_ref[...], s2_ref[...])
        o_ref[...] = (_bdot(jnp.tanh(acc), wc_ref[...]) + bc_ref[...])


def kernel(features, A0, W1, W2, Wc, bc):
    n, kdim = features.shape
    h = W1.shape[1]
    f = W2.shape[1]
    c = Wc.shape[1]
    bm = _pick_block(n, 400)
    nb = n // bm

    w1b = W1.astype(jnp.bfloat16)
    w2b = W2.astype(jnp.bfloat16)
    wcb = Wc.astype(jnp.bfloat16)

    s1 = pl.pallas_call(
        _s1_kernel,
        grid=(nb,),
        in_specs=[
            pl.BlockSpec((bm, kdim), lambda i: (i, 0)),
            pl.BlockSpec((kdim, h), lambda i: (0, 0)),
        ],
        out_specs=pl.BlockSpec((bm, h), lambda i: (i, 0)),
        out_shape=jax.ShapeDtypeStruct((n, h), jnp.bfloat16),
        compiler_params=pltpu.CompilerParams(
            dimension_semantics=("arbitrary",)),
    )(features, w1b)

    logits = pl.pallas_call(
        functools.partial(_fused_kernel, nb=nb, bm=bm),
        grid=(2 * nb,),
        in_specs=[
            pl.BlockSpec((bm, n), lambda i: (lax.rem(i, nb), 0)),
            pl.BlockSpec((n, h), lambda i: (0, 0)),
            pl.BlockSpec((h, f), lambda i: (0, 0)),
            pl.BlockSpec((f, c), lambda i: (0, 0)),
            pl.BlockSpec((1, c), lambda i: (0, 0)),
        ],
        out_specs=pl.BlockSpec(
            (bm, c), lambda i: (jnp.where(i < nb, 0, i - nb), 0)),
        out_shape=jax.ShapeDtypeStruct((n, c), jnp.float32),
        scratch_shapes=[pltpu.VMEM((n, f), jnp.bfloat16)],
        compiler_params=pltpu.CompilerParams(
            dimension_semantics=("arbitrary",)),
    )(A0, s1, w2b, wcb, bc.reshape(1, -1))

    return logits
